# baseline (speedup 1.0000x reference)
import jax
import jax.numpy as jnp
from jax import lax
from jax.experimental import pallas as pl
from jax.experimental.pallas import tpu as pltpu
from jax.experimental.pallas import tpu_sc as plsc

_INFO = plsc.get_sparse_core_info()
_NC = _INFO.num_cores
_NS = _INFO.num_subcores
_NW = _NC * _NS
_L = _INFO.num_lanes

_B = 16384
_EMB = 64
_BPW = _B // _NW
_CHUNK = 128
_NCHUNK = _BPW // _CHUNK


def _mf_kernel(u_ids, i_ids, user_emb, user_bias, item_emb, item_bias, mean,
               out, uidx_v, iidx_v, urows_v, irows_v, ubias_v, ibias_v,
               mean_v, out_v, sem):
    wid = lax.axis_index("s") * _NC + lax.axis_index("c")
    base = wid * _BPW

    pltpu.sync_copy(u_ids.at[pl.ds(base, _BPW)], uidx_v)
    pltpu.sync_copy(i_ids.at[pl.ds(base, _BPW)], iidx_v)
    pltpu.sync_copy(mean, mean_v)

    copies = []
    for j in range(_NCHUNK):
        sl = pl.ds(j * _CHUNK, _CHUNK)
        copies.append(pltpu.async_copy(
            user_emb.at[uidx_v.at[sl]], urows_v.at[sl], sem))
        copies.append(pltpu.async_copy(
            item_emb.at[iidx_v.at[sl]], irows_v.at[sl], sem))
        copies.append(pltpu.async_copy(
            user_bias.at[uidx_v.at[sl]], ubias_v.at[sl], sem))
        copies.append(pltpu.async_copy(
            item_bias.at[iidx_v.at[sl]], ibias_v.at[sl], sem))
    for c in copies:
        c.wait()

    mean_vec = mean_v[pl.ds(0, _L)]
    iota16 = lax.iota(jnp.int32, _L)

    def group_body(k, carry):
        row_idx = k * _L + iota16
        acc = jnp.zeros((_L,), jnp.float32)
        for j in range(_EMB):
            colj = jnp.full((_L,), j, jnp.int32)
            u = plsc.load_gather(urows_v, [row_idx, colj])
            w = plsc.load_gather(irows_v, [row_idx, colj])
            acc = acc + u * w
        sl = pl.ds(k * _L, _L)
        v = acc + ubias_v[sl] + ibias_v[sl] + mean_vec
        out_v[sl] = 1.0 / (1.0 + jnp.exp(-v))
        return carry

    lax.fori_loop(0, _BPW // _L, group_body, 0)

    pltpu.sync_copy(out_v, out.at[pl.ds(base, _BPW)])


@jax.jit
def kernel(fields, user_emb, user_bias, item_emb, item_bias, mean):
    u_ids = fields[:, 0]
    i_ids = fields[:, 1]
    mean16 = jnp.broadcast_to(mean, (_L,))
    mesh = plsc.VectorSubcoreMesh(core_axis_name="c", subcore_axis_name="s")
    run = pl.kernel(
        _mf_kernel,
        out_type=jax.ShapeDtypeStruct((_B,), jnp.float32),
        mesh=mesh,
        scratch_types=[
            pltpu.VMEM((_BPW,), jnp.int32),
            pltpu.VMEM((_BPW,), jnp.int32),
            pltpu.VMEM((_BPW, _EMB), jnp.float32),
            pltpu.VMEM((_BPW, _EMB), jnp.float32),
            pltpu.VMEM((_BPW,), jnp.float32),
            pltpu.VMEM((_BPW,), jnp.float32),
            pltpu.VMEM((_L,), jnp.float32),
            pltpu.VMEM((_BPW,), jnp.float32),
            pltpu.SemaphoreType.DMA,
        ],
        compiler_params=pltpu.CompilerParams(
            needs_layout_passes=False, use_tc_tiling_on_sc=False),
    )
    return run(u_ids, i_ids, user_emb.reshape(-1, _EMB),
               user_bias.reshape(-1), item_emb.reshape(-1, _EMB),
               item_bias.reshape(-1), mean16)


# speedup vs baseline: 2.3269x; 2.3269x over previous
import jax
import jax.numpy as jnp
from jax import lax
from jax.experimental import pallas as pl
from jax.experimental.pallas import tpu as pltpu
from jax.experimental.pallas import tpu_sc as plsc

_INFO = plsc.get_sparse_core_info()
_NC = _INFO.num_cores
_NS = _INFO.num_subcores
_NW = _NC * _NS
_L = _INFO.num_lanes

_B = 16384
_EMB = 64
_N = 1000000
_BPW = _B // _NW
_CHUNK = 128
_NCHUNK = _BPW // _CHUNK
_D = 4
_MAXOFF = _N - _CHUNK


def _mf_kernel(u_ids, i_ids, uT, ubias, iT, ibias, mean, out,
               uidx_v, iidx_v, ublk, iblk, ubias_v, ibias_v,
               mean_v, acc_v, out_v, sem_b,
               us0, us1, us2, us3, is0, is1, is2, is3):
    usem = (us0, us1, us2, us3)
    isem = (is0, is1, is2, is3)
    wid = lax.axis_index("s") * _NC + lax.axis_index("c")
    base = wid * _BPW

    pltpu.sync_copy(u_ids.at[pl.ds(base, _BPW)], uidx_v)
    pltpu.sync_copy(i_ids.at[pl.ds(base, _BPW)], iidx_v)
    pltpu.sync_copy(mean, mean_v)

    bias_copies = []
    for j in range(_NCHUNK):
        sl = pl.ds(j * _CHUNK, _CHUNK)
        bias_copies.append(pltpu.async_copy(
            ubias.at[uidx_v.at[sl]], ubias_v.at[sl], sem_b))
        bias_copies.append(pltpu.async_copy(
            ibias.at[iidx_v.at[sl]], ibias_v.at[sl], sem_b))

    iota16 = lax.iota(jnp.int32, _L)

    def sidx(ref, r):
        v = ref[pl.ds((r // _L) * _L, _L)]
        dnums = lax.GatherDimensionNumbers(
            offset_dims=(), collapsed_slice_dims=(0,), start_index_map=(0,))
        g = lax.gather(v, jnp.full((_L, 1), r % _L, jnp.int32), dnums, (1,),
                       mode=lax.GatherScatterMode.PROMISE_IN_BOUNDS)
        return g[0]

    def blk_off(idx):
        return jnp.minimum((idx // _CHUNK) * _CHUNK, _MAXOFF)

    def fire(r, slot):
        uo = blk_off(sidx(uidx_v, r))
        pltpu.async_copy(uT.at[:, pl.ds(pl.multiple_of(uo, _CHUNK), _CHUNK)],
                         ublk.at[slot], usem[slot])
        io = blk_off(sidx(iidx_v, r))
        pltpu.async_copy(iT.at[:, pl.ds(pl.multiple_of(io, _CHUNK), _CHUNK)],
                         iblk.at[slot], isem[slot])

    def consume(r, slot):
        u = sidx(uidx_v, r)
        i = sidx(iidx_v, r)
        cu = jnp.full((_L,), u - blk_off(u), jnp.int32)
        ci = jnp.full((_L,), i - blk_off(i), jnp.int32)
        pltpu.make_async_copy(uT.at[:, pl.ds(0, _CHUNK)], ublk.at[slot],
                              usem[slot]).wait()
        pltpu.make_async_copy(iT.at[:, pl.ds(0, _CHUNK)], iblk.at[slot],
                              isem[slot]).wait()
        rows = iota16
        uv = plsc.load_gather(ublk.at[slot], [rows, cu])
        iv = plsc.load_gather(iblk.at[slot], [rows, ci])
        acc = uv * iv
        for g in range(1, _EMB // _L):
            rows = g * _L + iota16
            uv = plsc.load_gather(ublk.at[slot], [rows, cu])
            iv = plsc.load_gather(iblk.at[slot], [rows, ci])
            acc = acc + uv * iv
        acc_v[pl.ds(r * _L, _L)] = acc

    for d in range(_D):
        fire(d, d)

    def ring_body(n, carry):
        for d in range(_D):
            r = n * _D + d
            consume(r, d)
            fire(r + _D, d)
        return carry

    lax.fori_loop(0, _BPW // _D - 1, ring_body, 0)
    for d in range(_D):
        consume(_BPW - _D + d, d)

    for c in bias_copies:
        c.wait()
    mean_vec = mean_v[pl.ds(0, _L)]

    def reduce_body(k, carry):
        flat = (k * _L + iota16) * _L
        s = plsc.load_gather(acc_v, [flat])
        for j in range(1, _L):
            s = s + plsc.load_gather(acc_v, [flat + j])
        sl = pl.ds(k * _L, _L)
        v = s + ubias_v[sl] + ibias_v[sl] + mean_vec
        out_v[sl] = 1.0 / (1.0 + jnp.exp(-v))
        return carry

    lax.fori_loop(0, _BPW // _L, reduce_body, 0)

    pltpu.sync_copy(out_v, out.at[pl.ds(base, _BPW)])


@jax.jit
def kernel(fields, user_emb, user_bias, item_emb, item_bias, mean):
    u_ids = fields[:, 0]
    i_ids = fields[:, 1]
    mean16 = jnp.broadcast_to(mean, (_L,))
    mesh = plsc.VectorSubcoreMesh(core_axis_name="c", subcore_axis_name="s")
    run = pl.kernel(
        _mf_kernel,
        out_type=jax.ShapeDtypeStruct((_B,), jnp.float32),
        mesh=mesh,
        scratch_types=[
            pltpu.VMEM((_BPW,), jnp.int32),
            pltpu.VMEM((_BPW,), jnp.int32),
            pltpu.VMEM((_D, _EMB, _CHUNK), jnp.float32),
            pltpu.VMEM((_D, _EMB, _CHUNK), jnp.float32),
            pltpu.VMEM((_BPW,), jnp.float32),
            pltpu.VMEM((_BPW,), jnp.float32),
            pltpu.VMEM((_L,), jnp.float32),
            pltpu.VMEM((_BPW * _L,), jnp.float32),
            pltpu.VMEM((_BPW,), jnp.float32),
            pltpu.SemaphoreType.DMA,
            pltpu.SemaphoreType.DMA,
            pltpu.SemaphoreType.DMA,
            pltpu.SemaphoreType.DMA,
            pltpu.SemaphoreType.DMA,
            pltpu.SemaphoreType.DMA,
            pltpu.SemaphoreType.DMA,
            pltpu.SemaphoreType.DMA,
            pltpu.SemaphoreType.DMA,
        ],
        compiler_params=pltpu.CompilerParams(
            needs_layout_passes=False, use_tc_tiling_on_sc=True),
    )
    return run(u_ids, i_ids, user_emb.T, user_bias.reshape(-1),
               item_emb.T, item_bias.reshape(-1), mean16)


# speedup vs baseline: 2.9280x; 1.2584x over previous
import jax
import jax.numpy as jnp
from jax import lax
from jax.experimental import pallas as pl
from jax.experimental.pallas import tpu as pltpu
from jax.experimental.pallas import tpu_sc as plsc

_INFO = plsc.get_sparse_core_info()
_NC = _INFO.num_cores
_NS = _INFO.num_subcores
_NW = _NC * _NS
_L = _INFO.num_lanes

_B = 16384
_EMB = 64
_N = 1000000
_BPW = _B // _NW
_CHUNK = 128
_NCHUNK = _BPW // _CHUNK
_D = 6
_MAXOFF = _N - _CHUNK


def _dot_kernel(u_ids, i_ids, uT, iT, dots,
                uidx_v, iidx_v, ublk, iblk, acc_v, out_v, *sems):
    usem = sems[:_D]
    isem = sems[_D:]
    wid = lax.axis_index("s") * _NC + lax.axis_index("c")
    base = wid * _BPW

    pltpu.sync_copy(u_ids.at[pl.ds(base, _BPW)], uidx_v)
    pltpu.sync_copy(i_ids.at[pl.ds(base, _BPW)], iidx_v)

    iota16 = lax.iota(jnp.int32, _L)

    def sidx(ref, r):
        v = ref[pl.ds((r // _L) * _L, _L)]
        dnums = lax.GatherDimensionNumbers(
            offset_dims=(), collapsed_slice_dims=(0,), start_index_map=(0,))
        g = lax.gather(v, jnp.full((_L, 1), r % _L, jnp.int32), dnums, (1,),
                       mode=lax.GatherScatterMode.PROMISE_IN_BOUNDS)
        return g[0]

    def blk_off(idx):
        return jnp.minimum((idx // _CHUNK) * _CHUNK, _MAXOFF)

    def fire(r, slot):
        uo = blk_off(sidx(uidx_v, r))
        pltpu.async_copy(uT.at[:, pl.ds(pl.multiple_of(uo, _CHUNK), _CHUNK)],
                         ublk.at[slot], usem[slot])
        io = blk_off(sidx(iidx_v, r))
        pltpu.async_copy(iT.at[:, pl.ds(pl.multiple_of(io, _CHUNK), _CHUNK)],
                         iblk.at[slot], isem[slot])

    def consume(r, slot):
        u = sidx(uidx_v, r)
        i = sidx(iidx_v, r)
        cu = jnp.full((_L,), u - blk_off(u), jnp.int32)
        ci = jnp.full((_L,), i - blk_off(i), jnp.int32)
        pltpu.make_async_copy(uT.at[:, pl.ds(0, _CHUNK)], ublk.at[slot],
                              usem[slot]).wait()
        pltpu.make_async_copy(iT.at[:, pl.ds(0, _CHUNK)], iblk.at[slot],
                              isem[slot]).wait()
        rows = iota16
        uv = plsc.load_gather(ublk.at[slot], [rows, cu])
        iv = plsc.load_gather(iblk.at[slot], [rows, ci])
        acc = uv * iv
        for g in range(1, _EMB // _L):
            rows = g * _L + iota16
            uv = plsc.load_gather(ublk.at[slot], [rows, cu])
            iv = plsc.load_gather(iblk.at[slot], [rows, ci])
            acc = acc + uv * iv
        acc_v[pl.ds(r * _L, _L)] = acc

    for d in range(_D):
        fire(d, d)

    def ring_body(n, carry):
        for d in range(_D):
            r = n * _D + d
            consume(r, d)
            fire(r + _D, d)
        return carry

    steady = (_BPW - _D) // _D * _D
    lax.fori_loop(0, steady // _D, ring_body, 0)
    for r in range(steady, _BPW - _D):
        consume(r, r % _D)
        fire(r + _D, r % _D)
    for r in range(_BPW - _D, _BPW):
        consume(r, r % _D)

    def reduce_body(k, carry):
        flat = (k * _L + iota16) * _L
        s = plsc.load_gather(acc_v, [flat])
        for j in range(1, _L):
            s = s + plsc.load_gather(acc_v, [flat + j])
        out_v[pl.ds(k * _L, _L)] = s
        return carry

    lax.fori_loop(0, _BPW // _L, reduce_body, 0)

    pltpu.sync_copy(out_v, dots.at[pl.ds(base, _BPW)])


def _bias_kernel(dots, u_ids, i_ids, ubias, ibias, mean, out,
                 uidx_v, iidx_v, dots_v, ubias_v, ibias_v, mean_v, out_v,
                 sem_b):
    wid = lax.axis_index("s") * _NC + lax.axis_index("c")
    base = wid * _BPW

    pltpu.sync_copy(u_ids.at[pl.ds(base, _BPW)], uidx_v)
    pltpu.sync_copy(i_ids.at[pl.ds(base, _BPW)], iidx_v)
    pltpu.sync_copy(dots.at[pl.ds(base, _BPW)], dots_v)
    pltpu.sync_copy(mean, mean_v)

    bias_copies = []
    for j in range(_NCHUNK):
        sl = pl.ds(j * _CHUNK, _CHUNK)
        bias_copies.append(pltpu.async_copy(
            ubias.at[uidx_v.at[sl]], ubias_v.at[sl], sem_b))
        bias_copies.append(pltpu.async_copy(
            ibias.at[iidx_v.at[sl]], ibias_v.at[sl], sem_b))
    for c in bias_copies:
        c.wait()

    mean_vec = mean_v[pl.ds(0, _L)]

    def sig_body(k, carry):
        sl = pl.ds(k * _L, _L)
        v = dots_v[sl] + ubias_v[sl] + ibias_v[sl] + mean_vec
        out_v[sl] = 1.0 / (1.0 + jnp.exp(-v))
        return carry

    lax.fori_loop(0, _BPW // _L, sig_body, 0)

    pltpu.sync_copy(out_v, out.at[pl.ds(base, _BPW)])


@jax.jit
def kernel(fields, user_emb, user_bias, item_emb, item_bias, mean):
    u_ids = fields[:, 0]
    i_ids = fields[:, 1]
    mean16 = jnp.broadcast_to(mean, (_L,))
    mesh = plsc.VectorSubcoreMesh(core_axis_name="c", subcore_axis_name="s")
    run_dot = pl.kernel(
        _dot_kernel,
        out_type=jax.ShapeDtypeStruct((_B,), jnp.float32),
        mesh=mesh,
        scratch_types=[
            pltpu.VMEM((_BPW,), jnp.int32),
            pltpu.VMEM((_BPW,), jnp.int32),
            pltpu.VMEM((_D, _EMB, _CHUNK), jnp.float32),
            pltpu.VMEM((_D, _EMB, _CHUNK), jnp.float32),
            pltpu.VMEM((_BPW * _L,), jnp.float32),
            pltpu.VMEM((_BPW,), jnp.float32),
        ] + [pltpu.SemaphoreType.DMA] * (2 * _D),
        compiler_params=pltpu.CompilerParams(
            needs_layout_passes=False, use_tc_tiling_on_sc=True),
    )
    dots = run_dot(u_ids, i_ids, user_emb.T, item_emb.T)
    run_bias = pl.kernel(
        _bias_kernel,
        out_type=jax.ShapeDtypeStruct((_B,), jnp.float32),
        mesh=mesh,
        scratch_types=[
            pltpu.VMEM((_BPW,), jnp.int32),
            pltpu.VMEM((_BPW,), jnp.int32),
            pltpu.VMEM((_BPW,), jnp.float32),
            pltpu.VMEM((_BPW,), jnp.float32),
            pltpu.VMEM((_BPW,), jnp.float32),
            pltpu.VMEM((_L,), jnp.float32),
            pltpu.VMEM((_BPW,), jnp.float32),
            pltpu.SemaphoreType.DMA,
        ],
        compiler_params=pltpu.CompilerParams(use_tc_tiling_on_sc=False),
    )
    return run_bias(dots, u_ids, i_ids, user_bias.reshape(-1),
                    item_bias.reshape(-1), mean16)
